# Initial kernel scaffold; baseline (speedup 1.0000x reference)
import jax
import jax.numpy as jnp
from jax.experimental import pallas as pl


def kernel(Z_t, Z_prev, adj_param):
    raise NotImplementedError("write your pallas kernel here")


# speedup vs baseline: 8.6396x; 8.6396x over previous
import jax
import jax.numpy as jnp
from jax.experimental import pallas as pl
from jax.experimental.pallas import tpu as pltpu

N = 4096
D = 512
K = 8
BLOCK = 256
GRID = N // BLOCK


def _gc_kernel(adj_ref, zt_ref, anorm_ref, asp_ref, zhat_ref, lrec_ref, lsm_ref):
    i = pl.program_id(0)

    a = jnp.abs(adj_ref[...])
    amin = jnp.min(a, axis=-1, keepdims=True)
    amax = jnp.max(a, axis=-1, keepdims=True)
    d1 = amax - amin + 1e-8
    inv_d1 = 1.0 / d1
    anorm = (a - amin) * inv_d1
    anorm_ref[...] = anorm

    iota = jax.lax.broadcasted_iota(jnp.int32, a.shape, 1)
    cur = a
    sel = None
    for k in range(K):
        m = amax if k == 0 else jnp.max(cur, axis=-1, keepdims=True)
        eq = cur >= m
        fi = jnp.min(jnp.where(eq, iota, N), axis=-1, keepdims=True)
        hit = iota == fi
        sel = hit if sel is None else (sel | hit)
        if k < K - 1:
            cur = jnp.where(hit, -1.0, cur)

    asp = jnp.where(sel, anorm, 0.0)
    asp_ref[...] = asp

    zt = zt_ref[...]
    zhat = jnp.dot(asp, zt, preferred_element_type=jnp.float32)
    zhat_ref[...] = zhat

    zt_blk = zt_ref[pl.ds(i * BLOCK, BLOCK), :]
    diff = zhat - zt_blk
    lrec_part = jnp.sum(diff * diff, keepdims=True)

    inv_d2 = 1.0 / (amax + 1e-8)
    sdiff = (a - amin) * (inv_d1 - inv_d2)
    lsm_part = jnp.sum(sdiff * sdiff, keepdims=True)

    @pl.when(i == 0)
    def _init():
        lrec_ref[...] = lrec_part
        lsm_ref[...] = lsm_part

    @pl.when(i > 0)
    def _acc():
        lrec_ref[...] += lrec_part
        lsm_ref[...] += lsm_part


@jax.jit
def kernel(Z_t, Z_prev, adj_param):
    out_shapes = (
        jax.ShapeDtypeStruct((N, N), jnp.float32),
        jax.ShapeDtypeStruct((N, N), jnp.float32),
        jax.ShapeDtypeStruct((N, D), jnp.float32),
        jax.ShapeDtypeStruct((1, 1), jnp.float32),
        jax.ShapeDtypeStruct((1, 1), jnp.float32),
    )
    a_norm, a_sparse, z_hat, lrec, lsm = pl.pallas_call(
        _gc_kernel,
        grid=(GRID,),
        in_specs=[
            pl.BlockSpec((BLOCK, N), lambda i: (i, 0)),
            pl.BlockSpec((N, D), lambda i: (0, 0)),
        ],
        out_specs=(
            pl.BlockSpec((BLOCK, N), lambda i: (i, 0)),
            pl.BlockSpec((BLOCK, N), lambda i: (i, 0)),
            pl.BlockSpec((BLOCK, D), lambda i: (i, 0)),
            pl.BlockSpec((1, 1), lambda i: (0, 0)),
            pl.BlockSpec((1, 1), lambda i: (0, 0)),
        ),
        out_shape=out_shapes,
        compiler_params=pltpu.CompilerParams(
            dimension_semantics=("arbitrary",),
        ),
    )(adj_param, Z_t)

    L_reconstruct = lrec[0, 0] / (N * D)
    L_smooth = lsm[0, 0] / (N * N)
    return (z_hat, a_norm, a_sparse, a_sparse, L_reconstruct, L_smooth)


# speedup vs baseline: 9.0593x; 1.0486x over previous
import jax
import jax.numpy as jnp
from jax.experimental import pallas as pl
from jax.experimental.pallas import tpu as pltpu

N = 4096
D = 512
K = 8
BLOCK = 256
GRID = N // BLOCK


def _gc_kernel(adj_ref, zt_ref, anorm_ref, asp_ref, zhat_ref, lrec_ref, lsm_ref):
    i = pl.program_id(0)

    a = jnp.abs(adj_ref[...])
    amin = jnp.min(a, axis=-1, keepdims=True)
    amax = jnp.max(a, axis=-1, keepdims=True)
    d1 = amax - amin + 1e-8
    inv_d1 = 1.0 / d1
    anorm = (a - amin) * inv_d1
    anorm_ref[...] = anorm

    iota = jax.lax.broadcasted_iota(jnp.int32, a.shape, 1)
    cur = a
    sel = None
    for k in range(K):
        am = jnp.argmax(cur, axis=-1, keepdims=True)
        hit = iota == am
        sel = hit if sel is None else (sel | hit)
        if k < K - 1:
            cur = jnp.where(hit, -1.0, cur)

    asp = jnp.where(sel, anorm, 0.0)
    asp_ref[...] = asp

    zt = zt_ref[...]
    zhat = jnp.dot(asp, zt, preferred_element_type=jnp.float32)
    zhat_ref[...] = zhat

    zt_blk = zt_ref[pl.ds(i * BLOCK, BLOCK), :]
    diff = zhat - zt_blk
    lrec_part = jnp.sum(diff * diff, keepdims=True)

    d2 = amax + 1e-8
    c = 1.0 - d1 / d2
    row_sq = jnp.sum(anorm * anorm, axis=-1, keepdims=True)
    lsm_part = jnp.sum(c * c * row_sq, keepdims=True)

    @pl.when(i == 0)
    def _init():
        lrec_ref[...] = lrec_part
        lsm_ref[...] = lsm_part

    @pl.when(i > 0)
    def _acc():
        lrec_ref[...] += lrec_part
        lsm_ref[...] += lsm_part


@jax.jit
def kernel(Z_t, Z_prev, adj_param):
    out_shapes = (
        jax.ShapeDtypeStruct((N, N), jnp.float32),
        jax.ShapeDtypeStruct((N, N), jnp.float32),
        jax.ShapeDtypeStruct((N, D), jnp.float32),
        jax.ShapeDtypeStruct((1, 1), jnp.float32),
        jax.ShapeDtypeStruct((1, 1), jnp.float32),
    )
    a_norm, a_sparse, z_hat, lrec, lsm = pl.pallas_call(
        _gc_kernel,
        grid=(GRID,),
        in_specs=[
            pl.BlockSpec((BLOCK, N), lambda i: (i, 0)),
            pl.BlockSpec((N, D), lambda i: (0, 0)),
        ],
        out_specs=(
            pl.BlockSpec((BLOCK, N), lambda i: (i, 0)),
            pl.BlockSpec((BLOCK, N), lambda i: (i, 0)),
            pl.BlockSpec((BLOCK, D), lambda i: (i, 0)),
            pl.BlockSpec((1, 1), lambda i: (0, 0)),
            pl.BlockSpec((1, 1), lambda i: (0, 0)),
        ),
        out_shape=out_shapes,
        compiler_params=pltpu.CompilerParams(
            dimension_semantics=("arbitrary",),
        ),
    )(adj_param, Z_t)

    L_reconstruct = lrec[0, 0] / (N * D)
    L_smooth = lsm[0, 0] / (N * N)
    return (z_hat, a_norm, a_sparse, a_sparse, L_reconstruct, L_smooth)


# speedup vs baseline: 10.3852x; 1.1464x over previous
import jax
import jax.numpy as jnp
from jax.experimental import pallas as pl
from jax.experimental.pallas import tpu as pltpu

N = 4096
D = 512
K = 8
BLOCK = 256
GRID = N // BLOCK
CH = 8
NC = N // CH


def _gc_kernel(adj_ref, zt_ref, anorm_ref, asp_ref, zhat_ref, lrec_ref, lsm_ref):
    i = pl.program_id(0)

    a = jnp.abs(adj_ref[...])
    amin = jnp.min(a, axis=-1, keepdims=True)
    amax = jnp.max(a, axis=-1, keepdims=True)
    d1 = amax - amin + 1e-8
    inv_d1 = 1.0 / d1
    anorm = (a - amin) * inv_d1
    anorm_ref[...] = anorm

    M = a[:, 0:NC]
    for g in range(1, CH):
        M = jnp.maximum(M, a[:, g * NC:(g + 1) * NC])
    liota = jax.lax.broadcasted_iota(jnp.int32, (BLOCK, NC), 1)
    Mi = jnp.full((BLOCK, NC), N, jnp.int32)
    for g in range(CH - 1, -1, -1):
        hit_g = a[:, g * NC:(g + 1) * NC] >= M
        Mi = jnp.where(hit_g, liota + g * NC, Mi)
    curM = M
    fcs = []
    for k in range(K):
        mM = jnp.max(curM, axis=-1, keepdims=True)
        eqM = curM >= mM
        fig = jnp.min(jnp.where(eqM, Mi, N), axis=-1, keepdims=True)
        fcs.append(jnp.bitwise_and(fig, NC - 1))
        if k < K - 1:
            curM = jnp.where(eqM & (Mi == fig), -1.0, curM)

    fc_all = jnp.concatenate(fcs, axis=-1)
    pieces = []
    gps = []
    for g in range(CH):
        out_g = jnp.zeros((BLOCK, K), jnp.float32)
        for s in range(NC // 128):
            sub = a[:, g * NC + s * 128: g * NC + (s + 1) * 128]
            lidx = fc_all - s * 128
            valid = lidx.astype(jnp.uint32) < 128
            got = jnp.take_along_axis(sub, jnp.bitwise_and(lidx, 127), axis=-1)
            out_g = jnp.where(valid, got, out_g)
        pieces.append(out_g)
        gps.append(fc_all + g * NC)
    candv = jnp.concatenate(pieces, axis=-1)
    gidx = jnp.concatenate(gps, axis=-1)

    curc = candv
    for k in range(K):
        mc = jnp.max(curc, axis=-1, keepdims=True)
        eqc = curc >= mc
        fic = jnp.min(jnp.where(eqc, gidx, N), axis=-1, keepdims=True)
        if k < K - 1:
            curc = jnp.where(eqc & (gidx <= fic), -1.0, curc)

    iota = jax.lax.broadcasted_iota(jnp.int32, a.shape, 1)
    sel = (a > mc) | ((a == mc) & (iota <= fic))
    asp = jnp.where(sel, anorm, 0.0)
    asp_ref[...] = asp

    zt = zt_ref[...]
    zhat = jnp.dot(asp, zt, preferred_element_type=jnp.float32)
    zhat_ref[...] = zhat

    zt_blk = zt_ref[pl.ds(i * BLOCK, BLOCK), :]
    diff = zhat - zt_blk
    lrec_part = jnp.sum(diff * diff, keepdims=True)

    d2 = amax + 1e-8
    c = 1.0 - d1 / d2
    row_sq = jnp.sum(anorm * anorm, axis=-1, keepdims=True)
    lsm_part = jnp.sum(c * c * row_sq, keepdims=True)

    @pl.when(i == 0)
    def _init():
        lrec_ref[...] = lrec_part
        lsm_ref[...] = lsm_part

    @pl.when(i > 0)
    def _acc():
        lrec_ref[...] += lrec_part
        lsm_ref[...] += lsm_part


@jax.jit
def kernel(Z_t, Z_prev, adj_param):
    out_shapes = (
        jax.ShapeDtypeStruct((N, N), jnp.float32),
        jax.ShapeDtypeStruct((N, N), jnp.float32),
        jax.ShapeDtypeStruct((N, D), jnp.float32),
        jax.ShapeDtypeStruct((1, 1), jnp.float32),
        jax.ShapeDtypeStruct((1, 1), jnp.float32),
    )
    a_norm, a_sparse, z_hat, lrec, lsm = pl.pallas_call(
        _gc_kernel,
        grid=(GRID,),
        in_specs=[
            pl.BlockSpec((BLOCK, N), lambda i: (i, 0)),
            pl.BlockSpec((N, D), lambda i: (0, 0)),
        ],
        out_specs=(
            pl.BlockSpec((BLOCK, N), lambda i: (i, 0)),
            pl.BlockSpec((BLOCK, N), lambda i: (i, 0)),
            pl.BlockSpec((BLOCK, D), lambda i: (i, 0)),
            pl.BlockSpec((1, 1), lambda i: (0, 0)),
            pl.BlockSpec((1, 1), lambda i: (0, 0)),
        ),
        out_shape=out_shapes,
        compiler_params=pltpu.CompilerParams(
            dimension_semantics=("arbitrary",),
        ),
    )(adj_param, Z_t)

    L_reconstruct = lrec[0, 0] / (N * D)
    L_smooth = lsm[0, 0] / (N * N)
    return (z_hat, a_norm, a_sparse, a_sparse, L_reconstruct, L_smooth)


# speedup vs baseline: 12.3747x; 1.1916x over previous
import jax
import jax.numpy as jnp
from jax.experimental import pallas as pl
from jax.experimental.pallas import tpu as pltpu

N = 4096
D = 512
K = 8
BLOCK = 256
GRID = N // BLOCK
CH = 8
NC = N // CH


def _gc_kernel(adj_ref, zt_ref, anorm_ref, asp_ref, phi_ref, zhat_ref, lrec_ref, lsm_ref):
    i = pl.program_id(0)

    a = jnp.abs(adj_ref[...])
    amin = jnp.min(a, axis=-1, keepdims=True)
    amax = jnp.max(a, axis=-1, keepdims=True)
    d1 = amax - amin + 1e-8
    inv_d1 = 1.0 / d1
    anorm = (a - amin) * inv_d1
    anorm_ref[...] = anorm

    M = a[:, 0:NC]
    for g in range(1, CH):
        M = jnp.maximum(M, a[:, g * NC:(g + 1) * NC])
    liota = jax.lax.broadcasted_iota(jnp.int32, (BLOCK, NC), 1)
    Mi = jnp.full((BLOCK, NC), N, jnp.int32)
    for g in range(CH - 1, -1, -1):
        hit_g = a[:, g * NC:(g + 1) * NC] >= M
        Mi = jnp.where(hit_g, liota + g * NC, Mi)
    curM = M
    fcs = []
    for k in range(K):
        mM = jnp.max(curM, axis=-1, keepdims=True)
        eqM = curM >= mM
        fig = jnp.min(jnp.where(eqM, Mi, N), axis=-1, keepdims=True)
        fcs.append(jnp.bitwise_and(fig, NC - 1))
        if k < K - 1:
            curM = jnp.where(eqM & (Mi == fig), -1.0, curM)

    fc_all = jnp.concatenate(fcs, axis=-1)
    pieces = []
    gps = []
    for g in range(CH):
        out_g = jnp.zeros((BLOCK, K), jnp.float32)
        for s in range(NC // 128):
            sub = a[:, g * NC + s * 128: g * NC + (s + 1) * 128]
            lidx = fc_all - s * 128
            valid = lidx.astype(jnp.uint32) < 128
            got = jnp.take_along_axis(sub, jnp.bitwise_and(lidx, 127), axis=-1)
            out_g = jnp.where(valid, got, out_g)
        pieces.append(out_g)
        gps.append(fc_all + g * NC)
    candv = jnp.concatenate(pieces, axis=-1)
    gidx = jnp.concatenate(gps, axis=-1)

    curc = candv
    for k in range(K):
        mc = jnp.max(curc, axis=-1, keepdims=True)
        eqc = curc >= mc
        fic = jnp.min(jnp.where(eqc, gidx, N), axis=-1, keepdims=True)
        if k < K - 1:
            curc = jnp.where(eqc & (gidx <= fic), -1.0, curc)

    iota = jax.lax.broadcasted_iota(jnp.int32, a.shape, 1)
    sel = (a > mc) | ((a == mc) & (iota <= fic))
    asp = jnp.where(sel, anorm, 0.0)
    asp_ref[...] = asp
    phi_ref[...] = asp

    zt = zt_ref[...]
    zhat = jnp.dot(asp, zt, preferred_element_type=jnp.float32)
    zhat_ref[...] = zhat

    zt_blk = zt_ref[pl.ds(i * BLOCK, BLOCK), :]
    diff = zhat - zt_blk
    lrec_part = jnp.sum(diff * diff, keepdims=True)

    d2 = amax + 1e-8
    c = 1.0 - d1 / d2
    row_sq = jnp.sum(anorm * anorm, axis=-1, keepdims=True)
    lsm_part = jnp.sum(c * c * row_sq, keepdims=True)

    @pl.when(i == 0)
    def _init():
        lrec_ref[...] = lrec_part
        lsm_ref[...] = lsm_part

    @pl.when(i > 0)
    def _acc():
        lrec_ref[...] += lrec_part
        lsm_ref[...] += lsm_part


@jax.jit
def kernel(Z_t, Z_prev, adj_param):
    out_shapes = (
        jax.ShapeDtypeStruct((N, N), jnp.float32),
        jax.ShapeDtypeStruct((N, N), jnp.float32),
        jax.ShapeDtypeStruct((N, N), jnp.float32),
        jax.ShapeDtypeStruct((N, D), jnp.float32),
        jax.ShapeDtypeStruct((1, 1), jnp.float32),
        jax.ShapeDtypeStruct((1, 1), jnp.float32),
    )
    a_norm, a_sparse, phi, z_hat, lrec, lsm = pl.pallas_call(
        _gc_kernel,
        grid=(GRID,),
        in_specs=[
            pl.BlockSpec((BLOCK, N), lambda i: (i, 0)),
            pl.BlockSpec((N, D), lambda i: (0, 0)),
        ],
        out_specs=(
            pl.BlockSpec((BLOCK, N), lambda i: (i, 0)),
            pl.BlockSpec((BLOCK, N), lambda i: (i, 0)),
            pl.BlockSpec((BLOCK, N), lambda i: (i, 0)),
            pl.BlockSpec((BLOCK, D), lambda i: (i, 0)),
            pl.BlockSpec((1, 1), lambda i: (0, 0)),
            pl.BlockSpec((1, 1), lambda i: (0, 0)),
        ),
        out_shape=out_shapes,
        compiler_params=pltpu.CompilerParams(
            dimension_semantics=("arbitrary",),
        ),
    )(adj_param, Z_t)

    L_reconstruct = lrec[0, 0] / (N * D)
    L_smooth = lsm[0, 0] / (N * N)
    return (z_hat, a_norm, a_sparse, phi, L_reconstruct, L_smooth)


# speedup vs baseline: 12.4554x; 1.0065x over previous
import jax
import jax.numpy as jnp
from jax.experimental import pallas as pl
from jax.experimental.pallas import tpu as pltpu

N = 4096
D = 512
K = 8
BLOCK = 256
GRID = N // BLOCK
CH = 8
NC = N // CH


def _gc_kernel(adj_ref, zt_ref, anorm_ref, asp_ref, phi_ref, zhat_ref, lrec_ref, lsm_ref):
    i = pl.program_id(0)

    a = jnp.abs(adj_ref[...])
    amin = jnp.min(a, axis=-1, keepdims=True)

    M = a[:, 0:NC]
    for g in range(1, CH):
        M = jnp.maximum(M, a[:, g * NC:(g + 1) * NC])
    liota = jax.lax.broadcasted_iota(jnp.int32, (BLOCK, NC), 1)
    Mi = jnp.full((BLOCK, NC), N, jnp.int32)
    for g in range(CH - 1, -1, -1):
        hit_g = a[:, g * NC:(g + 1) * NC] >= M
        Mi = jnp.where(hit_g, liota + g * NC, Mi)

    amax = jnp.max(M, axis=-1, keepdims=True)
    d1 = amax - amin + 1e-8
    inv_d1 = 1.0 / d1
    anorm = (a - amin) * inv_d1
    anorm_ref[...] = anorm

    curM = M
    fcs = []
    for k in range(K):
        mM = jnp.max(curM, axis=-1, keepdims=True)
        eqM = curM >= mM
        fig = jnp.min(jnp.where(eqM, Mi, N), axis=-1, keepdims=True)
        fcs.append(jnp.bitwise_and(fig, NC - 1))
        if k < K - 1:
            curM = jnp.where(eqM & (Mi == fig), -1.0, curM)

    fc_all = jnp.concatenate(fcs, axis=-1)
    pieces = []
    gps = []
    for g in range(CH):
        out_g = jnp.zeros((BLOCK, K), jnp.float32)
        for s in range(NC // 128):
            sub = a[:, g * NC + s * 128: g * NC + (s + 1) * 128]
            lidx = fc_all - s * 128
            valid = lidx.astype(jnp.uint32) < 128
            got = jnp.take_along_axis(sub, jnp.bitwise_and(lidx, 127), axis=-1)
            out_g = jnp.where(valid, got, out_g)
        pieces.append(out_g)
        gps.append(fc_all + g * NC)
    candv = jnp.concatenate(pieces, axis=-1)
    gidx = jnp.concatenate(gps, axis=-1)

    curc = candv
    for k in range(K):
        mc = jnp.max(curc, axis=-1, keepdims=True)
        eqc = curc >= mc
        fic = jnp.min(jnp.where(eqc, gidx, N), axis=-1, keepdims=True)
        if k < K - 1:
            curc = jnp.where(eqc & (gidx <= fic), -1.0, curc)

    iota = jax.lax.broadcasted_iota(jnp.int32, a.shape, 1)
    sel = (a > mc) | ((a == mc) & (iota <= fic))
    asp = jnp.where(sel, anorm, 0.0)
    asp_ref[...] = asp
    phi_ref[...] = asp

    zt = zt_ref[...]
    zhat = jnp.dot(asp, zt, preferred_element_type=jnp.float32)
    zhat_ref[...] = zhat

    zt_blk = zt_ref[pl.ds(i * BLOCK, BLOCK), :]
    diff = zhat - zt_blk
    lrec_part = jnp.sum(diff * diff, keepdims=True)

    d2 = amax + 1e-8
    c = 1.0 - d1 / d2
    row_sq = jnp.sum(anorm * anorm, axis=-1, keepdims=True)
    lsm_part = jnp.sum(c * c * row_sq, keepdims=True)

    lrec_ref[...] = lrec_part.reshape(1, 1, 1)
    lsm_ref[...] = lsm_part.reshape(1, 1, 1)


@jax.jit
def kernel(Z_t, Z_prev, adj_param):
    out_shapes = (
        jax.ShapeDtypeStruct((N, N), jnp.float32),
        jax.ShapeDtypeStruct((N, N), jnp.float32),
        jax.ShapeDtypeStruct((N, N), jnp.float32),
        jax.ShapeDtypeStruct((N, D), jnp.float32),
        jax.ShapeDtypeStruct((GRID, 1, 1), jnp.float32),
        jax.ShapeDtypeStruct((GRID, 1, 1), jnp.float32),
    )
    a_norm, a_sparse, phi, z_hat, lrec, lsm = pl.pallas_call(
        _gc_kernel,
        grid=(GRID,),
        in_specs=[
            pl.BlockSpec((BLOCK, N), lambda i: (i, 0)),
            pl.BlockSpec((N, D), lambda i: (0, 0)),
        ],
        out_specs=(
            pl.BlockSpec((BLOCK, N), lambda i: (i, 0)),
            pl.BlockSpec((BLOCK, N), lambda i: (i, 0)),
            pl.BlockSpec((BLOCK, N), lambda i: (i, 0)),
            pl.BlockSpec((BLOCK, D), lambda i: (i, 0)),
            pl.BlockSpec((1, 1, 1), lambda i: (i, 0, 0)),
            pl.BlockSpec((1, 1, 1), lambda i: (i, 0, 0)),
        ),
        out_shape=out_shapes,
        compiler_params=pltpu.CompilerParams(
            dimension_semantics=("parallel",),
        ),
    )(adj_param, Z_t)

    L_reconstruct = jnp.sum(lrec) / (N * D)
    L_smooth = jnp.sum(lsm) / (N * N)
    return (z_hat, a_norm, a_sparse, phi, L_reconstruct, L_smooth)
